# baseline (speedup 1.0000x reference)
import functools

import jax
import jax.numpy as jnp
from jax import lax
from jax.experimental import pallas as pl
from jax.experimental.pallas import tpu as pltpu
from jax.experimental.pallas import tpu_sc as plsc

V = 1000000
NW = 32
NFULL = 3906
NB_MINE = NFULL // NW
NB_REST = NFULL - NB_MINE * NW


def _make_transpose():
    mesh = plsc.VectorSubcoreMesh(core_axis_name="c", subcore_axis_name="s")

    @functools.partial(
        pl.kernel,
        mesh=mesh,
        compiler_params=pltpu.CompilerParams(
            use_tc_tiling_on_sc=True, needs_layout_passes=False
        ),
        out_type=jax.ShapeDtypeStruct((V, 128), jnp.float32),
        scratch_types=[
            [pltpu.VMEM((64, 256), jnp.float32) for _ in range(2)],
            [pltpu.VMEM((256, 128), jnp.float32) for _ in range(2)],
            pltpu.VMEM((64, 64), jnp.float32),
            pltpu.VMEM((64, 128), jnp.float32),
            [pltpu.SemaphoreType.DMA for _ in range(2)],
            [pltpu.SemaphoreType.DMA for _ in range(2)],
        ],
    )
    def kt(t_hbm, tpad_hbm, tins, touts, tin_tail, tout_tail, isems, osems):
        wid = lax.axis_index("s") * 2 + lax.axis_index("c")
        n_mine = NB_MINE + jnp.where(wid < NB_REST, 1, 0)

        def blk(i):
            return i * NW + wid

        def in_descs(i, slot):
            base = pl.multiple_of(blk(i) * 256, 256)
            return [
                pltpu.make_async_copy(
                    t_hbm.at[pl.ds(fb * 8, 8), pl.ds(base, 256)],
                    tins[slot].at[pl.ds(fb * 8, 8)],
                    isems[slot],
                )
                for fb in range(8)
            ]

        def in_start(i, slot):
            for d in in_descs(i, slot):
                d.start()

        def in_wait(i, slot):
            for d in in_descs(i, slot):
                d.wait()

        def out_desc(i, slot):
            base = pl.multiple_of(blk(i) * 256, 256)
            return pltpu.make_async_copy(
                touts[slot], tpad_hbm.at[pl.ds(base, 256)], osems[slot]
            )

        def permute(tin, tout):
            dvecs = [k * 16 + lax.iota(jnp.int32, 16) for k in range(4)]

            def out_row(l, carry):
                vals = [
                    plsc.load_gather(tin, [dvecs[k], jnp.full((16,), l, jnp.int32)])
                    for k in range(4)
                ]
                for k in range(4):
                    tout[l, pl.ds(k * 16, 16)] = vals[k]
                return carry

            lax.fori_loop(0, 256, out_row, 0)

        in_start(0, 0)

        def pair(g, carry):
            for s in range(2):
                i = g * 2 + s

                @pl.when(i < n_mine)
                def _step():
                    in_wait(i, s)

                    @pl.when(i + 1 < n_mine)
                    def _next_in():
                        in_start(i + 1, 1 - s)

                    @pl.when(i >= 2)
                    def _drain():
                        out_desc(i - 2, s).wait()

                    permute(tins[s], touts[s])
                    out_desc(i, s).start()

            return carry

        lax.fori_loop(0, (NB_MINE + 2) // 2, pair, 0)

        @pl.when(lax.rem(n_mine, 2) == 0)
        def _drain_even():
            out_desc(n_mine - 2, 0).wait()
            out_desc(n_mine - 1, 1).wait()

        @pl.when(lax.rem(n_mine, 2) == 1)
        def _drain_odd():
            out_desc(n_mine - 2, 1).wait()
            out_desc(n_mine - 1, 0).wait()

        @pl.when(wid == 4)
        def _tail():
            pltpu.sync_copy(t_hbm.at[:, pl.ds(NFULL * 256, 64)], tin_tail)
            dvecs = [k * 16 + lax.iota(jnp.int32, 16) for k in range(4)]

            def tail_row(l, carry):
                vals = [
                    plsc.load_gather(
                        tin_tail, [dvecs[k], jnp.full((16,), l, jnp.int32)]
                    )
                    for k in range(4)
                ]
                for k in range(4):
                    tout_tail[l, pl.ds(k * 16, 16)] = vals[k]
                return carry

            lax.fori_loop(0, 64, tail_row, 0)
            pltpu.sync_copy(tout_tail, tpad_hbm.at[pl.ds(NFULL * 256, 64)])

    return kt


def _make_gather(B: int):
    b_per_w = B // NW
    CH = 200
    NBUF = 4
    LOOKAHEAD = 2
    n_ch = b_per_w // CH

    mesh = plsc.VectorSubcoreMesh(core_axis_name="c", subcore_axis_name="s")

    @functools.partial(
        pl.kernel,
        mesh=mesh,
        compiler_params=pltpu.CompilerParams(
            use_tc_tiling_on_sc=True, needs_layout_passes=False
        ),
        out_type=jax.ShapeDtypeStruct((4096, 200, 128), jnp.float32),
        scratch_types=[
            pltpu.VMEM((b_per_w,), jnp.int32),
            [pltpu.VMEM((CH, 128), jnp.float32) for _ in range(NBUF)],
            [pltpu.SemaphoreType.DMA for _ in range(NBUF)],
            [pltpu.SemaphoreType.DMA for _ in range(NBUF)],
        ],
    )
    def kg(idx_hbm, tpad_hbm, out_hbm, idx_v, rows_v, gsems, ssems):
        wid = lax.axis_index("s") * 2 + lax.axis_index("c")
        base = pl.multiple_of(wid * b_per_w, b_per_w)
        b0_base = wid * 128
        pltpu.sync_copy(idx_hbm.at[pl.ds(base, b_per_w)], idx_v)

        def gather_desc(c, slot):
            off = pl.multiple_of(c * CH, CH)
            return pltpu.make_async_copy(
                tpad_hbm.at[idx_v.at[pl.ds(off, CH)]], rows_v[slot], gsems[slot]
            )

        def store_desc(c, slot):
            return pltpu.make_async_copy(
                rows_v[slot], out_hbm.at[b0_base + c], ssems[slot]
            )

        for b in range(LOOKAHEAD):
            gather_desc(b, b).start()

        def group(g, carry):
            for b in range(NBUF):
                c = g * NBUF + b
                gather_desc(c, b).wait()
                store_desc(c, b).start()
                nxt_slot = (b + LOOKAHEAD) % NBUF
                nxt = c + LOOKAHEAD

                @pl.when(nxt < n_ch)
                def _issue_next():
                    @pl.when(nxt >= NBUF)
                    def _drain_prev():
                        store_desc(nxt - NBUF, nxt_slot).wait()

                    gather_desc(nxt, nxt_slot).start()

            return carry

        lax.fori_loop(0, n_ch // NBUF, group, 0)
        for b in range(NBUF):
            store_desc(n_ch - NBUF + b, (n_ch - NBUF + b) % NBUF).wait()

    return kg


def kernel(x, table):
    B = x.shape[0] * x.shape[1]
    tableT = jnp.swapaxes(table, 0, 1)
    tpad = _make_transpose()(tableT)
    idx = x.reshape(B)
    return _make_gather(B)(idx, tpad)[:, :, :64]


# speedup vs baseline: 1.4216x; 1.4216x over previous
import functools

import jax
import jax.numpy as jnp
from jax import lax
from jax.experimental import pallas as pl
from jax.experimental.pallas import tpu as pltpu
from jax.experimental.pallas import tpu_sc as plsc

D_MODEL = 64


def _make_gather(B: int, V: int, D: int):
    NW = 32
    assert B % NW == 0
    b_per_w = B // NW
    CH = 400
    NBUF = 4
    LOOKAHEAD = 2
    assert b_per_w % (CH * NBUF) == 0
    n_ch = b_per_w // CH

    mesh = plsc.VectorSubcoreMesh(core_axis_name="c", subcore_axis_name="s")

    @functools.partial(
        pl.kernel,
        mesh=mesh,
        compiler_params=pltpu.CompilerParams(use_tc_tiling_on_sc=False),
        out_type=jax.ShapeDtypeStruct((B, D), jnp.float32),
        scratch_types=[
            pltpu.VMEM((b_per_w,), jnp.int32),
            [pltpu.VMEM((CH, D), jnp.float32) for _ in range(NBUF)],
            [pltpu.SemaphoreType.DMA for _ in range(NBUF)],
            [pltpu.SemaphoreType.DMA for _ in range(NBUF)],
        ],
    )
    def k(idx_hbm, table_hbm, out_hbm, idx_v, rows_v, gsems, ssems):
        wid = lax.axis_index("s") * 2 + lax.axis_index("c")
        base = pl.multiple_of(wid * b_per_w, b_per_w)
        pltpu.sync_copy(idx_hbm.at[pl.ds(base, b_per_w)], idx_v)

        def gather_desc(c, slot):
            off = pl.multiple_of(c * CH, CH)
            return pltpu.make_async_copy(
                table_hbm.at[idx_v.at[pl.ds(off, CH)]], rows_v[slot], gsems[slot]
            )

        def store_desc(c, slot):
            off = pl.multiple_of(base + c * CH, CH)
            return pltpu.make_async_copy(
                rows_v[slot], out_hbm.at[pl.ds(off, CH)], ssems[slot]
            )

        for b in range(LOOKAHEAD):
            gather_desc(b, b).start()

        def group(g, carry):
            for b in range(NBUF):
                c = g * NBUF + b
                gather_desc(c, b).wait()
                store_desc(c, b).start()

                nxt_slot = (b + LOOKAHEAD) % NBUF
                nxt = c + LOOKAHEAD

                @pl.when(nxt < n_ch)
                def _issue_next():
                    @pl.when(nxt >= NBUF)
                    def _drain_prev():
                        store_desc(nxt - NBUF, nxt_slot).wait()

                    gather_desc(nxt, nxt_slot).start()

            return carry

        lax.fori_loop(0, n_ch // NBUF, group, 0)

        for b in range(NBUF):
            c = n_ch - NBUF + b
            store_desc(c, (n_ch - NBUF + b) % NBUF).wait()

    return k


def kernel(x, table):
    B = x.shape[0] * x.shape[1]
    idx = x.reshape(B).astype(jnp.int32)
    out = _make_gather(B, table.shape[0], table.shape[1])(idx, table)
    return out.reshape(x.shape[0], x.shape[1], table.shape[1])
